# baseline (speedup 1.0000x reference)
import jax
import jax.numpy as jnp
from jax import lax
from jax.experimental import pallas as pl
from jax.experimental.pallas import tpu as pltpu

N_ITEMS = 10000
N_BASKETS = 2000
D = 128

CB = 80
NB = N_BASKETS // CB
RB = 4
DB = 3
NCACHE = 19
CA = 200
NA = N_ITEMS // CA
RA = 4
DA = 3

F32 = jnp.float32
BF16 = jnp.bfloat16


def _mega_kernel(x16_ref, a_hbm, b_hbm, item_hbm, basket_ref,
                 b16c, land_b, land_a, b1_16, i1_16, bsum16, stage,
                 bsem, asem, osem):

    def b_cp(i, slot):
        return pltpu.make_async_copy(
            b_hbm.at[pl.ds(pl.multiple_of(i * CB, 8), CB), :], land_b.at[slot], bsem.at[slot])

    def a_cp(i, slot):
        return pltpu.make_async_copy(
            a_hbm.at[pl.ds(pl.multiple_of(i * CA, 8), CA), :], land_a.at[slot], asem.at[slot])

    def o_cp(i, slot):
        return pltpu.make_async_copy(
            stage.at[slot], item_hbm.at[pl.ds(pl.multiple_of(i * CA, 8), CA), :], osem.at[slot])

    for k in range(DB):
        b_cp(k, k).start()

    def p0(i, _):
        slot = lax.rem(i, RB)

        @pl.when(i + DB < NB)
        def _():
            b_cp(i + DB, lax.rem(i + DB, RB)).start()

        b_cp(i, slot).wait()
        b16 = land_b[slot].astype(BF16)

        @pl.when(i < NCACHE)
        def _():
            b16c[pl.ds(pl.multiple_of(i * CB, 16), CB), :] = b16

        b1c = jnp.dot(b16, x16_ref[...], preferred_element_type=F32)
        b1_16[pl.ds(pl.multiple_of(i * CB, 16), CB), :] = b1c.astype(BF16)
        return 0

    lax.fori_loop(0, NB, p0, 0)
    for t in range(min(RB, NB - NCACHE)):
        b_cp(NCACHE + t, t).start()

    for k in range(DA):
        a_cp(k, k).start()

    def p1(i, _):
        slot = lax.rem(i, RA)

        @pl.when(i + DA < NA)
        def _():
            a_cp(i + DA, lax.rem(i + DA, RA)).start()

        a_cp(i, slot).wait()
        a16 = land_a[slot].astype(BF16)
        i1c = jnp.dot(a16, b1_16[...], preferred_element_type=F32)
        i1_16[pl.ds(pl.multiple_of(i * CA, 16), CA), :] = i1c.astype(BF16)
        return 0

    lax.fori_loop(0, NA, p1, 0)

    for k in range(DA):
        a_cp(k, k).start()

    def p2(j, _):
        off = pl.multiple_of(j * CB, 16)
        b2c = jnp.dot(b16c[pl.ds(off, CB), :], i1_16[...],
                      preferred_element_type=F32)
        bsc = b1_16[pl.ds(off, CB), :].astype(F32) + b2c
        basket_ref[pl.ds(off, CB), :] = bsc * 0.5
        bsum16[pl.ds(off, CB), :] = bsc.astype(BF16)
        return 0

    lax.fori_loop(0, NCACHE, p2, 0)
    for t in range(NB - NCACHE):
        b_cp(NCACHE + t, t % RB).wait()
        tail16 = land_b[t % RB].astype(BF16)
        b2_t = jnp.dot(tail16, i1_16[...], preferred_element_type=F32)
        off = (NCACHE + t) * CB
        bs_t = b1_16[off:off + CB, :].astype(F32) + b2_t
        basket_ref[off:off + CB, :] = bs_t * 0.5
        bsum16[off:off + CB, :] = bs_t.astype(BF16)
        if t + RB < NB - NCACHE:
            b_cp(NCACHE + t + RB, t % RB).start()

    def p3(i, _):
        slot = lax.rem(i, RA)
        oslot = lax.rem(i, 2)

        @pl.when(i + DA < NA)
        def _():
            a_cp(i + DA, lax.rem(i + DA, RA)).start()

        a_cp(i, slot).wait()
        a16 = land_a[slot].astype(BF16)
        i12 = jnp.dot(a16, bsum16[...], preferred_element_type=F32)

        @pl.when(i >= 2)
        def _():
            o_cp(i - 2, oslot).wait()

        x32 = x16_ref[pl.ds(pl.multiple_of(i * CA, 16), CA), :].astype(F32)
        stage[pl.ds(oslot, 1), :, :] = ((x32 + i12) * (1.0 / 3.0))[None]
        o_cp(i, oslot).start()
        return 0

    lax.fori_loop(0, NA, p3, 0)
    o_cp(NA - 2, lax.rem(NA - 2, 2)).wait()
    o_cp(NA - 1, lax.rem(NA - 1, 2)).wait()


@jax.jit
def kernel(input, coef_item_rep, coef_basket_rep):
    x16 = input.astype(BF16)
    item_out, basket_out = pl.pallas_call(
        _mega_kernel,
        grid=(1,),
        in_specs=[
            pl.BlockSpec((N_ITEMS, D), lambda i: (0, 0)),
            pl.BlockSpec(memory_space=pltpu.MemorySpace.HBM),
            pl.BlockSpec(memory_space=pltpu.MemorySpace.HBM),
        ],
        out_specs=[
            pl.BlockSpec(memory_space=pltpu.MemorySpace.HBM),
            pl.BlockSpec((N_BASKETS, D), lambda i: (0, 0)),
        ],
        out_shape=[
            jax.ShapeDtypeStruct((N_ITEMS, D), F32),
            jax.ShapeDtypeStruct((N_BASKETS, D), F32),
        ],
        scratch_shapes=[
            pltpu.VMEM((NCACHE * CB, N_ITEMS), BF16),
            pltpu.VMEM((RB, CB, N_ITEMS), F32),
            pltpu.VMEM((RA, CA, N_BASKETS), F32),
            pltpu.VMEM((N_BASKETS, D), BF16),
            pltpu.VMEM((N_ITEMS, D), BF16),
            pltpu.VMEM((N_BASKETS, D), BF16),
            pltpu.VMEM((2, CA, D), F32),
            pltpu.SemaphoreType.DMA((RB,)),
            pltpu.SemaphoreType.DMA((RA,)),
            pltpu.SemaphoreType.DMA((2,)),
        ],
        compiler_params=pltpu.CompilerParams(
            dimension_semantics=("arbitrary",)),
    )(x16, coef_item_rep, coef_basket_rep)
    return (item_out, basket_out)


# speedup vs baseline: 1.0034x; 1.0034x over previous
import jax
import jax.numpy as jnp
from jax import lax
from jax.experimental import pallas as pl
from jax.experimental.pallas import tpu as pltpu

N_ITEMS = 10000
N_BASKETS = 2000
D = 128

CB = 80
NB = N_BASKETS // CB
RB = 4
DB = 3
NCACHE = 19
CA = 200
NA = N_ITEMS // CA
RA = 4
DA = 3

F32 = jnp.float32
BF16 = jnp.bfloat16


def _mega_kernel(x16_ref, a_hbm, b_hbm, item_hbm, basket_ref,
                 b16c, land_b, land_a, b1_16, i1_16, bsum16, stage,
                 bsem, asem, osem):

    def b_cp(i, slot):
        return pltpu.make_async_copy(
            b_hbm.at[pl.ds(pl.multiple_of(i * CB, 8), CB), :], land_b.at[slot], bsem.at[slot])

    def a_cp(i, slot):
        return pltpu.make_async_copy(
            a_hbm.at[pl.ds(pl.multiple_of(i * CA, 8), CA), :], land_a.at[slot], asem.at[slot])

    def o_cp(i, slot):
        return pltpu.make_async_copy(
            stage.at[slot], item_hbm.at[pl.ds(pl.multiple_of(i * CA, 8), CA), :], osem.at[slot])

    for k in range(DB):
        b_cp(k, k).start()

    def p0(i, _):
        slot = lax.rem(i, RB)

        @pl.when(i + DB < NB)
        def _():
            b_cp(i + DB, lax.rem(i + DB, RB)).start()

        b_cp(i, slot).wait()

        @pl.when(i < NCACHE)
        def _():
            b16c[pl.ds(pl.multiple_of(i * CB, 16), CB), :] = \
                land_b[slot].astype(BF16)

        b1c = jnp.dot(land_b[slot], x16_ref[...].astype(F32),
                      preferred_element_type=F32)
        b1_16[pl.ds(pl.multiple_of(i * CB, 16), CB), :] = b1c.astype(BF16)
        return 0

    lax.fori_loop(0, NB, p0, 0)
    for t in range(min(RB, NB - NCACHE)):
        b_cp(NCACHE + t, t).start()

    for k in range(DA):
        a_cp(k, k).start()

    def p1(i, _):
        slot = lax.rem(i, RA)

        @pl.when(i + DA < NA)
        def _():
            a_cp(i + DA, lax.rem(i + DA, RA)).start()

        a_cp(i, slot).wait()
        i1c = jnp.dot(land_a[slot], b1_16[...].astype(F32),
                      preferred_element_type=F32)
        i1_16[pl.ds(pl.multiple_of(i * CA, 16), CA), :] = i1c.astype(BF16)
        return 0

    lax.fori_loop(0, NA, p1, 0)

    for k in range(DA):
        a_cp(k, k).start()

    def p2(j, _):
        off = pl.multiple_of(j * CB, 16)
        b2c = jnp.dot(b16c[pl.ds(off, CB), :], i1_16[...],
                      preferred_element_type=F32)
        bsc = b1_16[pl.ds(off, CB), :].astype(F32) + b2c
        basket_ref[pl.ds(off, CB), :] = bsc * 0.5
        bsum16[pl.ds(off, CB), :] = bsc.astype(BF16)
        return 0

    lax.fori_loop(0, NCACHE, p2, 0)
    for t in range(NB - NCACHE):
        b_cp(NCACHE + t, t % RB).wait()
        b2_t = jnp.dot(land_b[t % RB], i1_16[...].astype(F32),
                       preferred_element_type=F32)
        off = (NCACHE + t) * CB
        bs_t = b1_16[off:off + CB, :].astype(F32) + b2_t
        basket_ref[off:off + CB, :] = bs_t * 0.5
        bsum16[off:off + CB, :] = bs_t.astype(BF16)
        if t + RB < NB - NCACHE:
            b_cp(NCACHE + t + RB, t % RB).start()

    def p3(i, _):
        slot = lax.rem(i, RA)
        oslot = lax.rem(i, 2)

        @pl.when(i + DA < NA)
        def _():
            a_cp(i + DA, lax.rem(i + DA, RA)).start()

        a_cp(i, slot).wait()
        i12 = jnp.dot(land_a[slot], bsum16[...].astype(F32),
                      preferred_element_type=F32)

        @pl.when(i >= 2)
        def _():
            o_cp(i - 2, oslot).wait()

        x32 = x16_ref[pl.ds(pl.multiple_of(i * CA, 16), CA), :].astype(F32)
        stage[pl.ds(oslot, 1), :, :] = ((x32 + i12) * (1.0 / 3.0))[None]
        o_cp(i, oslot).start()
        return 0

    lax.fori_loop(0, NA, p3, 0)
    o_cp(NA - 2, lax.rem(NA - 2, 2)).wait()
    o_cp(NA - 1, lax.rem(NA - 1, 2)).wait()


@jax.jit
def kernel(input, coef_item_rep, coef_basket_rep):
    x16 = input.astype(BF16)
    item_out, basket_out = pl.pallas_call(
        _mega_kernel,
        grid=(1,),
        in_specs=[
            pl.BlockSpec((N_ITEMS, D), lambda i: (0, 0)),
            pl.BlockSpec(memory_space=pltpu.MemorySpace.HBM),
            pl.BlockSpec(memory_space=pltpu.MemorySpace.HBM),
        ],
        out_specs=[
            pl.BlockSpec(memory_space=pltpu.MemorySpace.HBM),
            pl.BlockSpec((N_BASKETS, D), lambda i: (0, 0)),
        ],
        out_shape=[
            jax.ShapeDtypeStruct((N_ITEMS, D), F32),
            jax.ShapeDtypeStruct((N_BASKETS, D), F32),
        ],
        scratch_shapes=[
            pltpu.VMEM((NCACHE * CB, N_ITEMS), BF16),
            pltpu.VMEM((RB, CB, N_ITEMS), F32),
            pltpu.VMEM((RA, CA, N_BASKETS), F32),
            pltpu.VMEM((N_BASKETS, D), BF16),
            pltpu.VMEM((N_ITEMS, D), BF16),
            pltpu.VMEM((N_BASKETS, D), BF16),
            pltpu.VMEM((2, CA, D), F32),
            pltpu.SemaphoreType.DMA((RB,)),
            pltpu.SemaphoreType.DMA((RA,)),
            pltpu.SemaphoreType.DMA((2,)),
        ],
        compiler_params=pltpu.CompilerParams(
            dimension_semantics=("arbitrary",)),
    )(x16, coef_item_rep, coef_basket_rep)
    return (item_out, basket_out)


# speedup vs baseline: 1.0311x; 1.0277x over previous
import jax
import jax.numpy as jnp
from jax import lax
from jax.experimental import pallas as pl
from jax.experimental.pallas import tpu as pltpu

N_ITEMS = 10000
N_BASKETS = 2000
D = 128

CB = 80
NB = N_BASKETS // CB
RB = 3
DB = 2
NCACHE = 21
CA = 400
NA = N_ITEMS // CA
RA = 4
DA = 3

F32 = jnp.float32
BF16 = jnp.bfloat16


def _mega_kernel(x16_ref, a_hbm, b_hbm, item_hbm, basket_ref,
                 b16c, land_b, land_a, b1_16, i1_16, bsum16, stage,
                 bsem, asem, osem):

    def b_cp(i, slot):
        return pltpu.make_async_copy(
            b_hbm.at[pl.ds(pl.multiple_of(i * CB, 8), CB), :], land_b.at[slot], bsem.at[slot])

    def a_cp(i, slot):
        return pltpu.make_async_copy(
            a_hbm.at[pl.ds(pl.multiple_of(i * CA, 8), CA), :], land_a.at[slot], asem.at[slot])

    def o_cp(i, slot):
        return pltpu.make_async_copy(
            stage.at[slot], item_hbm.at[pl.ds(pl.multiple_of(i * CA, 8), CA), :], osem.at[slot])

    for k in range(DB):
        b_cp(k, k).start()

    def p0(i, _):
        slot = lax.rem(i, RB)

        @pl.when(i + DB < NB)
        def _():
            b_cp(i + DB, lax.rem(i + DB, RB)).start()

        b_cp(i, slot).wait()

        @pl.when(i < NCACHE)
        def _():
            b16c[pl.ds(pl.multiple_of(i * CB, 16), CB), :] = \
                land_b[slot].astype(BF16)

        b1c = jnp.dot(land_b[slot], x16_ref[...].astype(F32),
                      preferred_element_type=F32)
        b1_16[pl.ds(pl.multiple_of(i * CB, 16), CB), :] = b1c.astype(BF16)
        return 0

    lax.fori_loop(0, NB, p0, 0)
    for t in range(min(RB, NB - NCACHE)):
        b_cp(NCACHE + t, t).start()

    for k in range(DA):
        a_cp(k, k).start()

    def p1(i, _):
        slot = lax.rem(i, RA)

        @pl.when(i + DA < NA)
        def _():
            a_cp(i + DA, lax.rem(i + DA, RA)).start()

        a_cp(i, slot).wait()
        i1c = jnp.dot(land_a[slot], b1_16[...].astype(F32),
                      preferred_element_type=F32)
        i1_16[pl.ds(pl.multiple_of(i * CA, 16), CA), :] = i1c.astype(BF16)
        return 0

    lax.fori_loop(0, NA, p1, 0)

    for k in range(DA):
        a_cp(k, k).start()

    def p2(j, _):
        off = pl.multiple_of(j * CB, 16)
        b2c = jnp.dot(b16c[pl.ds(off, CB), :], i1_16[...],
                      preferred_element_type=F32)
        bsc = b1_16[pl.ds(off, CB), :].astype(F32) + b2c
        basket_ref[pl.ds(off, CB), :] = bsc * 0.5
        bsum16[pl.ds(off, CB), :] = bsc.astype(BF16)
        return 0

    lax.fori_loop(0, NCACHE, p2, 0)
    for t in range(NB - NCACHE):
        b_cp(NCACHE + t, t % RB).wait()
        b2_t = jnp.dot(land_b[t % RB], i1_16[...].astype(F32),
                       preferred_element_type=F32)
        off = (NCACHE + t) * CB
        bs_t = b1_16[off:off + CB, :].astype(F32) + b2_t
        basket_ref[off:off + CB, :] = bs_t * 0.5
        bsum16[off:off + CB, :] = bs_t.astype(BF16)
        if t + RB < NB - NCACHE:
            b_cp(NCACHE + t + RB, t % RB).start()

    def p3(i, _):
        slot = lax.rem(i, RA)
        oslot = lax.rem(i, 2)

        @pl.when(i + DA < NA)
        def _():
            a_cp(i + DA, lax.rem(i + DA, RA)).start()

        a_cp(i, slot).wait()
        i12 = jnp.dot(land_a[slot], bsum16[...].astype(F32),
                      preferred_element_type=F32)

        @pl.when(i >= 2)
        def _():
            o_cp(i - 2, oslot).wait()

        x32 = x16_ref[pl.ds(pl.multiple_of(i * CA, 16), CA), :].astype(F32)
        stage[pl.ds(oslot, 1), :, :] = ((x32 + i12) * (1.0 / 3.0))[None]
        o_cp(i, oslot).start()
        return 0

    lax.fori_loop(0, NA, p3, 0)
    o_cp(NA - 2, lax.rem(NA - 2, 2)).wait()
    o_cp(NA - 1, lax.rem(NA - 1, 2)).wait()


@jax.jit
def kernel(input, coef_item_rep, coef_basket_rep):
    x16 = input.astype(BF16)
    item_out, basket_out = pl.pallas_call(
        _mega_kernel,
        grid=(1,),
        in_specs=[
            pl.BlockSpec((N_ITEMS, D), lambda i: (0, 0)),
            pl.BlockSpec(memory_space=pltpu.MemorySpace.HBM),
            pl.BlockSpec(memory_space=pltpu.MemorySpace.HBM),
        ],
        out_specs=[
            pl.BlockSpec(memory_space=pltpu.MemorySpace.HBM),
            pl.BlockSpec((N_BASKETS, D), lambda i: (0, 0)),
        ],
        out_shape=[
            jax.ShapeDtypeStruct((N_ITEMS, D), F32),
            jax.ShapeDtypeStruct((N_BASKETS, D), F32),
        ],
        scratch_shapes=[
            pltpu.VMEM((NCACHE * CB, N_ITEMS), BF16),
            pltpu.VMEM((RB, CB, N_ITEMS), F32),
            pltpu.VMEM((RA, CA, N_BASKETS), F32),
            pltpu.VMEM((N_BASKETS, D), BF16),
            pltpu.VMEM((N_ITEMS, D), BF16),
            pltpu.VMEM((N_BASKETS, D), BF16),
            pltpu.VMEM((2, CA, D), F32),
            pltpu.SemaphoreType.DMA((RB,)),
            pltpu.SemaphoreType.DMA((RA,)),
            pltpu.SemaphoreType.DMA((2,)),
        ],
        compiler_params=pltpu.CompilerParams(
            dimension_semantics=("arbitrary",),
            vmem_limit_bytes=64 * 1024 * 1024),
    )(x16, coef_item_rep, coef_basket_rep)
    return (item_out, basket_out)


# speedup vs baseline: 1.0320x; 1.0008x over previous
import jax
import jax.numpy as jnp
from jax import lax
from jax.experimental import pallas as pl
from jax.experimental.pallas import tpu as pltpu

N_ITEMS = 10000
N_BASKETS = 2000
D = 128

CB = 80
NB = N_BASKETS // CB
NCACHE = 20
CA = 400
NA = N_ITEMS // CA

F32 = jnp.float32
BF16 = jnp.bfloat16


def _mega_kernel(x16_ref, a_hbm, b_hbm, item_hbm, basket_ref,
                 b16c, lbe, lbo, lae, lao, b1_16, i1_16, bsum16, stage,
                 sbe, sbo, sae, sao, osem):
    def cpb(c, ring_ref, sem_ref, slot):
        return pltpu.make_async_copy(
            b_hbm.at[pl.ds(pl.multiple_of(c * CB, 8), CB), :],
            ring_ref.at[slot], sem_ref.at[slot])

    def cpa(c, ring_ref, sem_ref, slot):
        return pltpu.make_async_copy(
            a_hbm.at[pl.ds(pl.multiple_of(c * CA, 8), CA), :],
            ring_ref.at[slot], sem_ref.at[slot])

    def o_cp(i, slot):
        return pltpu.make_async_copy(
            stage.at[slot],
            item_hbm.at[pl.ds(pl.multiple_of(i * CA, 8), CA), :],
            osem.at[slot])

    def consume_b0(c, ring_ref, slot):
        @pl.when(c < NCACHE)
        def _():
            b16c[pl.ds(pl.multiple_of(c * CB, 16), CB), :] = \
                ring_ref[slot].astype(BF16)
        b1c = jnp.dot(ring_ref[slot], x16_ref[...].astype(F32),
                      preferred_element_type=F32)
        b1_16[pl.ds(pl.multiple_of(c * CB, 16), CB), :] = b1c.astype(BF16)

    cpb(0, lbe, sbe, 0).start()
    cpb(1, lbo, sbo, 0).start()
    cpb(2, lbe, sbe, 1).start()
    cpb(3, lbo, sbo, 1).start()

    def p0(j, _):
        slot = lax.rem(j, 2)
        ce = 2 * j
        cpb(ce, lbe, sbe, slot).wait()
        consume_b0(ce, lbe, slot)

        @pl.when(ce + 4 < NB)
        def _():
            cpb(ce + 4, lbe, sbe, slot).start()

        co = 2 * j + 1
        cpb(co, lbo, sbo, slot).wait()
        consume_b0(co, lbo, slot)

        @pl.when(co + 4 < NB)
        def _():
            cpb(co + 4, lbo, sbo, slot).start()
        return 0

    lax.fori_loop(0, NB // 2, p0, 0)
    cpb(NB - 1, lbe, sbe, 0).wait()
    consume_b0(NB - 1, lbe, 0)

    cpb(NCACHE, lbe, sbe, 0).start()
    cpb(NCACHE + 1, lbo, sbo, 0).start()
    cpb(NCACHE + 2, lbe, sbe, 1).start()
    cpb(NCACHE + 3, lbo, sbo, 1).start()

    def consume_a1(c, ring_ref, slot):
        i1c = jnp.dot(ring_ref[slot], b1_16[...].astype(F32),
                      preferred_element_type=F32)
        i1_16[pl.ds(pl.multiple_of(c * CA, 16), CA), :] = i1c.astype(BF16)

    cpa(0, lae, sae, 0).start()
    cpa(1, lao, sao, 0).start()
    cpa(2, lae, sae, 1).start()
    cpa(3, lao, sao, 1).start()

    def p1(j, _):
        slot = lax.rem(j, 2)
        ce = 2 * j
        cpa(ce, lae, sae, slot).wait()
        consume_a1(ce, lae, slot)

        @pl.when(ce + 4 < NA)
        def _():
            cpa(ce + 4, lae, sae, slot).start()

        co = 2 * j + 1
        cpa(co, lao, sao, slot).wait()
        consume_a1(co, lao, slot)

        @pl.when(co + 4 < NA)
        def _():
            cpa(co + 4, lao, sao, slot).start()
        return 0

    lax.fori_loop(0, NA // 2, p1, 0)
    cpa(NA - 1, lae, sae, 0).wait()
    consume_a1(NA - 1, lae, 0)

    def bask_chunk(c, b2c):
        off16 = pl.multiple_of(c * CB, 16)
        bsc = b1_16[pl.ds(off16, CB), :].astype(F32) + b2c
        basket_ref[pl.ds(off16, CB), :] = bsc * 0.5
        bsum16[pl.ds(off16, CB), :] = bsc.astype(BF16)

    def p2(j, _):
        off16 = pl.multiple_of(j * CB, 16)
        b2c = jnp.dot(b16c[pl.ds(off16, CB), :], i1_16[...].astype(F32),
                      preferred_element_type=F32)
        bask_chunk(j, b2c)
        return 0

    lax.fori_loop(0, NCACHE, p2, 0)

    def tail_dot(ring_ref, slot):
        return jnp.dot(ring_ref[slot], i1_16[...].astype(F32),
                       preferred_element_type=F32)

    cpb(NCACHE, lbe, sbe, 0).wait()
    bask_chunk(NCACHE, tail_dot(lbe, 0))
    cpb(NB - 1, lbe, sbe, 0).start()
    cpb(NCACHE + 1, lbo, sbo, 0).wait()
    bask_chunk(NCACHE + 1, tail_dot(lbo, 0))
    cpb(NCACHE + 2, lbe, sbe, 1).wait()
    bask_chunk(NCACHE + 2, tail_dot(lbe, 1))
    cpb(NCACHE + 3, lbo, sbo, 1).wait()
    bask_chunk(NCACHE + 3, tail_dot(lbo, 1))
    cpb(NB - 1, lbe, sbe, 0).wait()
    bask_chunk(NB - 1, tail_dot(lbe, 0))

    cpa(0, lae, sae, 0).start()
    cpa(1, lao, sao, 0).start()
    cpa(2, lae, sae, 1).start()
    cpa(3, lao, sao, 1).start()

    def consume_a3(c, ring_ref, slot):
        i12 = jnp.dot(ring_ref[slot], bsum16[...].astype(F32),
                      preferred_element_type=F32)
        oslot = lax.rem(c, 2)

        @pl.when(c >= 2)
        def _():
            o_cp(c - 2, oslot).wait()

        x32 = x16_ref[pl.ds(pl.multiple_of(c * CA, 16), CA), :].astype(F32)
        stage[pl.ds(oslot, 1), :, :] = ((x32 + i12) * (1.0 / 3.0))[None]
        o_cp(c, oslot).start()

    def p3(j, _):
        slot = lax.rem(j, 2)
        ce = 2 * j
        cpa(ce, lae, sae, slot).wait()
        consume_a3(ce, lae, slot)

        @pl.when(ce + 4 < NA)
        def _():
            cpa(ce + 4, lae, sae, slot).start()

        co = 2 * j + 1
        cpa(co, lao, sao, slot).wait()
        consume_a3(co, lao, slot)

        @pl.when(co + 4 < NA)
        def _():
            cpa(co + 4, lao, sao, slot).start()
        return 0

    lax.fori_loop(0, NA // 2, p3, 0)
    cpa(NA - 1, lae, sae, 0).wait()
    consume_a3(NA - 1, lae, 0)
    o_cp(NA - 2, lax.rem(NA - 2, 2)).wait()
    o_cp(NA - 1, lax.rem(NA - 1, 2)).wait()


@jax.jit
def kernel(input, coef_item_rep, coef_basket_rep):
    x16 = input.astype(BF16)
    item_out, basket_out = pl.pallas_call(
        _mega_kernel,
        grid=(1,),
        in_specs=[
            pl.BlockSpec((N_ITEMS, D), lambda i: (0, 0)),
            pl.BlockSpec(memory_space=pltpu.MemorySpace.HBM),
            pl.BlockSpec(memory_space=pltpu.MemorySpace.HBM),
        ],
        out_specs=[
            pl.BlockSpec(memory_space=pltpu.MemorySpace.HBM),
            pl.BlockSpec((N_BASKETS, D), lambda i: (0, 0)),
        ],
        out_shape=[
            jax.ShapeDtypeStruct((N_ITEMS, D), F32),
            jax.ShapeDtypeStruct((N_BASKETS, D), F32),
        ],
        scratch_shapes=[
            pltpu.VMEM((NCACHE * CB, N_ITEMS), BF16),
            pltpu.VMEM((2, CB, N_ITEMS), F32),
            pltpu.VMEM((2, CB, N_ITEMS), F32),
            pltpu.VMEM((2, CA, N_BASKETS), F32),
            pltpu.VMEM((2, CA, N_BASKETS), F32),
            pltpu.VMEM((N_BASKETS, D), BF16),
            pltpu.VMEM((N_ITEMS, D), BF16),
            pltpu.VMEM((N_BASKETS, D), BF16),
            pltpu.VMEM((2, CA, D), F32),
            pltpu.SemaphoreType.DMA((2,)),
            pltpu.SemaphoreType.DMA((2,)),
            pltpu.SemaphoreType.DMA((2,)),
            pltpu.SemaphoreType.DMA((2,)),
            pltpu.SemaphoreType.DMA((2,)),
        ],
        compiler_params=pltpu.CompilerParams(
            dimension_semantics=("arbitrary",),
            vmem_limit_bytes=64 * 1024 * 1024),
    )(x16, coef_item_rep, coef_basket_rep)
    return (item_out, basket_out)


# speedup vs baseline: 1.0349x; 1.0029x over previous
import jax
import jax.numpy as jnp
from jax.experimental import pallas as pl
from jax.experimental.pallas import tpu as pltpu

N_ITEMS = 10000
N_BASKETS = 2000
D = 128

BRB = 200
ARB = 1000
NCROWS = 1000
P1, P2, P3, NSTEPS = 10, 20, 25, 35

F32 = jnp.float32
BF16 = jnp.bfloat16


def _fused_kernel(x16_ref, a_ref, b_ref, item_ref, basket_ref,
                  b16c, b1_16, i1_16, bsum16):
    p = pl.program_id(0)

    @pl.when(p < P1)
    def _phase0():
        s = p

        @pl.when(s < NCROWS // BRB)
        def _():
            b16c[pl.ds(pl.multiple_of(s * BRB, 16), BRB), :] = \
                b_ref[...].astype(BF16)

        b1c = jnp.dot(b_ref[...], x16_ref[...].astype(F32),
                      preferred_element_type=F32)
        b1_16[pl.ds(pl.multiple_of(s * BRB, 16), BRB), :] = b1c.astype(BF16)

    @pl.when((p >= P1) & (p < P2))
    def _phase1():
        s = p - P1
        i1c = jnp.dot(a_ref[...], b1_16[...].astype(F32),
                      preferred_element_type=F32)
        i1_16[pl.ds(pl.multiple_of(s * ARB, 16), ARB), :] = i1c.astype(BF16)

    @pl.when((p >= P2) & (p < P3))
    def _phase2():
        s = p - P2
        i1f = i1_16[...].astype(F32)
        off_lo = pl.multiple_of(s * BRB, 16)
        b2lo = jnp.dot(b16c[pl.ds(off_lo, BRB), :], i1f,
                       preferred_element_type=F32)
        bslo = b1_16[pl.ds(off_lo, BRB), :].astype(F32) + b2lo
        basket_ref[pl.ds(off_lo, BRB), :] = bslo * 0.5
        bsum16[pl.ds(off_lo, BRB), :] = bslo.astype(BF16)
        off_hi = pl.multiple_of(NCROWS + s * BRB, 16)
        b2hi = jnp.dot(b_ref[...], i1f, preferred_element_type=F32)
        bshi = b1_16[pl.ds(off_hi, BRB), :].astype(F32) + b2hi
        basket_ref[pl.ds(off_hi, BRB), :] = bshi * 0.5
        bsum16[pl.ds(off_hi, BRB), :] = bshi.astype(BF16)

    @pl.when(p >= P3)
    def _phase3():
        s = p - P3
        i12 = jnp.dot(a_ref[...], bsum16[...].astype(F32),
                      preferred_element_type=F32)
        x32 = x16_ref[pl.ds(pl.multiple_of(s * ARB, 16), ARB), :].astype(F32)
        item_ref[...] = (x32 + i12) * (1.0 / 3.0)


def _b_index(p):
    return (jnp.where(p < P1, p,
                      jnp.where(p < P2, P1 - 1,
                                jnp.clip(NCROWS // BRB + (p - P2), 0,
                                         N_BASKETS // BRB - 1))), 0)


def _a_index(p):
    return (jnp.where(p < P2, jnp.clip(p - P1, 0, N_ITEMS // ARB - 1),
                      jnp.clip(p - P3, 0, N_ITEMS // ARB - 1)), 0)


def _item_index(p):
    return (jnp.clip(p - P3, 0, N_ITEMS // ARB - 1), 0)


@jax.jit
def kernel(input, coef_item_rep, coef_basket_rep):
    x16 = input.astype(BF16)
    item_out, basket_out = pl.pallas_call(
        _fused_kernel,
        grid=(NSTEPS,),
        in_specs=[
            pl.BlockSpec((N_ITEMS, D), lambda p: (0, 0)),
            pl.BlockSpec((ARB, N_BASKETS), _a_index),
            pl.BlockSpec((BRB, N_ITEMS), _b_index),
        ],
        out_specs=[
            pl.BlockSpec((ARB, D), _item_index),
            pl.BlockSpec((N_BASKETS, D), lambda p: (0, 0)),
        ],
        out_shape=[
            jax.ShapeDtypeStruct((N_ITEMS, D), F32),
            jax.ShapeDtypeStruct((N_BASKETS, D), F32),
        ],
        scratch_shapes=[
            pltpu.VMEM((NCROWS, N_ITEMS), BF16),
            pltpu.VMEM((N_BASKETS, D), BF16),
            pltpu.VMEM((N_ITEMS, D), BF16),
            pltpu.VMEM((N_BASKETS, D), BF16),
        ],
        compiler_params=pltpu.CompilerParams(
            dimension_semantics=("arbitrary",),
            vmem_limit_bytes=64 * 1024 * 1024),
    )(x16, coef_item_rep, coef_basket_rep)
    return (item_out, basket_out)
